# baseline (speedup 1.0000x reference)
import dataclasses

import jax
import jax.numpy as jnp
from jax import lax
from jax.experimental import pallas as pl
from jax.experimental.pallas import tpu as pltpu
from jax.experimental.pallas import tpu_sc as plsc

_NUM_CORES = 2
_NUM_SUBCORES = 16
_NUM_WORKERS = _NUM_CORES * _NUM_SUBCORES
_LANES = 16


def kernel(session_week_id, emb_weight):
    batch = session_week_id.shape[0]
    num_rows, dim = emb_weight.shape
    b_per_w = batch // _NUM_WORKERS
    idx = session_week_id.astype(jnp.int32)
    table_m = emb_weight.reshape(num_rows // 2, 2 * dim)

    n_chunks = 4
    chunk = b_per_w // n_chunks

    mesh = plsc.VectorSubcoreMesh(core_axis_name="c", subcore_axis_name="s")

    compiler_params = pltpu.CompilerParams()
    if "needs_layout_passes" in pltpu.CompilerParams.__dataclass_fields__:
        compiler_params = dataclasses.replace(
            compiler_params, needs_layout_passes=False
        )

    @pl.kernel(
        out_type=jax.ShapeDtypeStruct((dim, batch), emb_weight.dtype),
        mesh=mesh,
        compiler_params=compiler_params,
        scratch_types=[
            pltpu.VMEM((b_per_w,), jnp.int32),
            pltpu.VMEM(table_m.shape, emb_weight.dtype),
            pltpu.VMEM((dim, chunk), emb_weight.dtype),
            pltpu.VMEM((dim, chunk), emb_weight.dtype),
            pltpu.SemaphoreType.DMA,
            pltpu.SemaphoreType.DMA,
        ],
    )
    def _gather(table_hbm, idx_hbm, out_hbm, idx_v, table_v, t0, t1, w0, w1):
        wid = lax.axis_index("s") * _NUM_CORES + lax.axis_index("c")
        base = wid * b_per_w
        pltpu.sync_copy(idx_hbm.at[pl.ds(base, b_per_w)], idx_v)
        pltpu.sync_copy(table_hbm, table_v)
        bufs_t = (t0, t1)
        wsems = (w0, w1)
        write_handles = [None, None]
        for k in range(n_chunks):
            b = k % 2
            if write_handles[b] is not None:
                write_handles[b].wait()
            t_ref = bufs_t[b]

            @pl.loop(0, chunk // _LANES)
            def _(i):
                idx16 = idx_v[pl.ds(k * chunk + i * _LANES, _LANES)]
                row16 = lax.shift_right_logical(idx16, 1)
                lane16 = lax.bitwise_and(idx16, 1) * dim
                for d in range(dim):
                    t_ref[d, pl.ds(i * _LANES, _LANES)] = plsc.load_gather(
                        table_v, [row16, lane16 + d]
                    )

            write_handles[b] = pltpu.async_copy(
                t_ref, out_hbm.at[:, pl.ds(base + k * chunk, chunk)], wsems[b]
            )
        write_handles[0].wait()
        write_handles[1].wait()

    return _gather(table_m, idx).T


# speedup vs baseline: 1.5686x; 1.5686x over previous
import jax
import jax.numpy as jnp
from jax import lax
from jax.experimental import pallas as pl
from jax.experimental.pallas import tpu as pltpu
from jax.experimental.pallas import tpu_sc as plsc

_NUM_CORES = 2
_NUM_SUBCORES = 16
_NUM_WORKERS = _NUM_CORES * _NUM_SUBCORES
_LANE_PAD = 128


def kernel(session_week_id, emb_weight):
    batch = session_week_id.shape[0]
    dim = emb_weight.shape[1]
    b_per_w = batch // _NUM_WORKERS
    idx = session_week_id.astype(jnp.int32)
    table = jnp.pad(emb_weight, ((0, 0), (0, _LANE_PAD - dim)))

    mesh = plsc.VectorSubcoreMesh(core_axis_name="c", subcore_axis_name="s")

    @pl.kernel(
        out_type=jax.ShapeDtypeStruct((batch, _LANE_PAD), emb_weight.dtype),
        mesh=mesh,
        scratch_types=[
            pltpu.VMEM((b_per_w,), jnp.int32),
            pltpu.VMEM((b_per_w, _LANE_PAD), emb_weight.dtype),
            pltpu.SemaphoreType.DMA,
        ],
    )
    def _gather(table_hbm, idx_hbm, out_hbm, idx_v, rows_v, sem):
        wid = lax.axis_index("s") * _NUM_CORES + lax.axis_index("c")
        base = wid * b_per_w
        pltpu.sync_copy(idx_hbm.at[pl.ds(base, b_per_w)], idx_v)
        pltpu.async_copy(table_hbm.at[idx_v], rows_v, sem).wait()
        pltpu.sync_copy(rows_v, out_hbm.at[pl.ds(base, b_per_w)])

    return _gather(table, idx)[:, :dim]
